# baseline (speedup 1.0000x reference)
import jax
import jax.numpy as jnp
from jax import lax
from jax.experimental import pallas as pl
from jax.experimental.pallas import tpu as pltpu
from jax.experimental.pallas import tpu_sc as plsc

D = 128
N = 10000
E = 320000
NC = 2
NS = 16
NW = NC * NS
RW = D // NW
NP = RW // 2
CHUNK = 4000
NCHUNKS = E // CHUNK

_MASK_HI = -65536
_HALF = 0x8000
_FMAX = 3.0e38


def _sc_body(m1_hbm, m2_hbm, polar_hbm, ind1_hbm, ind2_hbm, out_hbm,
             tA0, tA1, tB0, tB1, rr0, rr1, rr2, rr3, idx1_v, idx2_v, out_v,
             sem_in0, sem_in1, sem_out0, sem_out1):
    tA = (tA0, tA1)
    tB = (tB0, tB1)
    rr = (rr0, rr1, rr2, rr3)
    sem_in = (sem_in0, sem_in1)
    sem_out = (sem_out0, sem_out1)

    wid = lax.axis_index("s") * NC + lax.axis_index("c")
    row0 = wid * RW

    def pack2(a, b):
        return plsc.bitcast(
            plsc.pack(a, b, format=plsc.PackFormat.INTERLEAVED), jnp.float32)

    for g in range(NP):
        r0 = row0 + 2 * g
        pltpu.sync_copy(m1_hbm.at[r0], tA[g])
        pltpu.sync_copy(m1_hbm.at[r0 + 1], tB[g])
        pltpu.sync_copy(polar_hbm.at[r0], rr[2 * g])
        pltpu.sync_copy(polar_hbm.at[r0 + 1], rr[2 * g + 1])
        pltpu.sync_copy(m2_hbm.at[r0], out_v.at[pl.ds(N, N)])
        pltpu.sync_copy(m2_hbm.at[r0 + 1], out_v.at[pl.ds(2 * N, N)])

        @plsc.parallel_loop(0, N, step=16, unroll=4)
        def prol(i, g=g):
            sl = pl.ds(i, 16)
            m1a = tA[g][sl]
            m1b = tB[g][sl]
            pa = rr[2 * g][sl]
            pb = rr[2 * g + 1][sl]
            m2a = out_v[pl.ds(N + i, 16)]
            m2b = out_v[pl.ds(2 * N + i, 16)]
            tA[g][sl] = pack2(m1a, m1b)
            tB[g][sl] = pack2(m2a, m2b)
            rr[2 * g][sl] = m1a / (1.5 * pa)
            rr[2 * g + 1][sl] = m1b / (1.5 * pb)

    def in_copies(c, p):
        e0 = c * CHUNK
        bsl = pl.ds(p * CHUNK, CHUNK)
        return (
            pltpu.make_async_copy(ind1_hbm.at[pl.ds(e0, CHUNK)],
                                  idx1_v.at[bsl], sem_in[p]),
            pltpu.make_async_copy(ind2_hbm.at[pl.ds(e0, CHUNK)],
                                  idx2_v.at[bsl], sem_in[p]),
        )

    def out_copies(c, p):
        e0 = c * CHUNK
        return tuple(
            pltpu.make_async_copy(
                out_v.at[pl.ds((p * RW + r) * CHUNK, CHUNK)],
                out_hbm.at[row0 + r, pl.ds(e0, CHUNK)],
                sem_out[p])
            for r in range(RW)
        )

    for cp in in_copies(0, 0):
        cp.start()

    def pair_body(c2, _):
        for p in (0, 1):
            c = c2 * 2 + p
            for cp in in_copies(c, p):
                cp.wait()
            @pl.when(c < NCHUNKS - 1)
            def _():
                for cp in in_copies(c + 1, 1 - p):
                    cp.start()
            @pl.when(c2 >= 1)
            def _():
                for cp in out_copies(c - 2, p):
                    cp.wait()

            @plsc.parallel_loop(0, CHUNK, step=16, unroll=2)
            def blk(e, p=p):
                i1 = idx1_v[pl.ds(p * CHUNK + e, 16)]
                i2 = idx2_v[pl.ds(p * CHUNK + e, 16)]
                for g in range(NP):
                    bf = jnp.bfloat16
                    xA1 = plsc.bitcast(plsc.load_gather(tA[g], [i1]), bf)
                    xA2 = plsc.bitcast(plsc.load_gather(tA[g], [i2]), bf)
                    xB1 = plsc.bitcast(plsc.load_gather(tB[g], [i1]), bf)
                    xB2 = plsc.bitcast(plsc.load_gather(tB[g], [i2]), bf)
                    nm = xA1 * xB2 + xA2 * xB1
                    n0, n1 = plsc.unpack(nm, format=plsc.PackFormat.INTERLEAVED)
                    for h, n in ((0, n0), (1, n1)):
                        r = 2 * g + h
                        q1 = plsc.load_gather(rr[r], [i1])
                        q2 = plsc.load_gather(rr[r], [i2])
                        out_v[pl.ds((p * RW + r) * CHUNK + e, 16)] = (
                            n / (q1 + q2))

            for cp in out_copies(c, p):
                cp.start()
        return 0

    lax.fori_loop(0, NCHUNKS // 2, pair_body, 0)

    for p in (0, 1):
        for cp in out_copies(NCHUNKS - 2 + p, p):
            cp.wait()


def kernel(m1, m2, polar, indices):
    ind1 = indices[0, :].astype(jnp.int32)
    ind2 = indices[1, :].astype(jnp.int32)
    mesh = plsc.VectorSubcoreMesh(core_axis_name="c", subcore_axis_name="s")
    f = pl.kernel(
        _sc_body,
        out_type=jax.ShapeDtypeStruct((D, E), jnp.float32),
        mesh=mesh,
        compiler_params=pltpu.CompilerParams(needs_layout_passes=False,
                                             use_tc_tiling_on_sc=False),
        scratch_types=(
            [pltpu.VMEM((N,), jnp.float32) for _ in range(2 * NP + RW)]
            + [
                pltpu.VMEM((2 * CHUNK,), jnp.int32),
                pltpu.VMEM((2 * CHUNK,), jnp.int32),
                pltpu.VMEM((2 * RW * CHUNK,), jnp.float32),
                pltpu.SemaphoreType.DMA,
                pltpu.SemaphoreType.DMA,
                pltpu.SemaphoreType.DMA,
                pltpu.SemaphoreType.DMA,
            ]
        ),
    )
    return f(m1, m2, polar, ind1, ind2)


# speedup vs baseline: 1.1223x; 1.1223x over previous
import jax
import jax.numpy as jnp
from jax import lax
from jax.experimental import pallas as pl
from jax.experimental.pallas import tpu as pltpu
from jax.experimental.pallas import tpu_sc as plsc

D = 128
N = 10000
E = 320000
NC = 2
NS = 16
NW = NC * NS
RW = D // NW
NP = RW // 2
CHUNK = 4000
NCHUNKS = E // CHUNK

_MASK_HI = -65536
_HALF = 0x8000
_FMAX = 3.0e38


def _sc_body(m1_hbm, m2_hbm, polar_hbm, ind1_hbm, ind2_hbm, out_hbm,
             tA0, tA1, tB0, tB1, tC0, tC1, idx1_v, idx2_v, out_v,
             sem_in0, sem_in1, sem_out0, sem_out1):
    tA = (tA0, tA1)
    tB = (tB0, tB1)
    tC = (tC0, tC1)
    sem_in = (sem_in0, sem_in1)
    sem_out = (sem_out0, sem_out1)

    wid = lax.axis_index("s") * NC + lax.axis_index("c")
    row0 = wid * RW

    def pack2(a, b):
        return plsc.bitcast(
            plsc.pack(a, b, format=plsc.PackFormat.INTERLEAVED), jnp.float32)

    for g in range(NP):
        r0 = row0 + 2 * g
        pltpu.sync_copy(m1_hbm.at[r0], tA[g])
        pltpu.sync_copy(m1_hbm.at[r0 + 1], tB[g])
        pltpu.sync_copy(polar_hbm.at[r0], tC[g])
        pltpu.sync_copy(polar_hbm.at[r0 + 1], out_v.at[pl.ds(0, N)])
        pltpu.sync_copy(m2_hbm.at[r0], out_v.at[pl.ds(N, N)])
        pltpu.sync_copy(m2_hbm.at[r0 + 1], out_v.at[pl.ds(2 * N, N)])

        @plsc.parallel_loop(0, N, step=16, unroll=4)
        def prol(i, g=g):
            sl = pl.ds(i, 16)
            m1a = tA[g][sl]
            m1b = tB[g][sl]
            pa = tC[g][sl]
            pb = out_v[sl]
            m2a = out_v[pl.ds(N + i, 16)]
            m2b = out_v[pl.ds(2 * N + i, 16)]
            tA[g][sl] = pack2(m1a, m1b)
            tB[g][sl] = pack2(m2a, m2b)
            tC[g][sl] = pack2(m1a / (1.5 * pa), m1b / (1.5 * pb))

    def in_copies(c, p):
        e0 = c * CHUNK
        bsl = pl.ds(p * CHUNK, CHUNK)
        return (
            pltpu.make_async_copy(ind1_hbm.at[pl.ds(e0, CHUNK)],
                                  idx1_v.at[bsl], sem_in[p]),
            pltpu.make_async_copy(ind2_hbm.at[pl.ds(e0, CHUNK)],
                                  idx2_v.at[bsl], sem_in[p]),
        )

    def out_copies(c, p):
        e0 = c * CHUNK
        return tuple(
            pltpu.make_async_copy(
                out_v.at[pl.ds((p * RW + r) * CHUNK, CHUNK)],
                out_hbm.at[row0 + r, pl.ds(e0, CHUNK)],
                sem_out[p])
            for r in range(RW)
        )

    for cp in in_copies(0, 0):
        cp.start()

    def pair_body(c2, _):
        for p in (0, 1):
            c = c2 * 2 + p
            for cp in in_copies(c, p):
                cp.wait()
            @pl.when(c < NCHUNKS - 1)
            def _():
                for cp in in_copies(c + 1, 1 - p):
                    cp.start()
            @pl.when(c2 >= 1)
            def _():
                for cp in out_copies(c - 2, p):
                    cp.wait()

            @plsc.parallel_loop(0, CHUNK, step=16, unroll=2)
            def blk(e, p=p):
                i1 = idx1_v[pl.ds(p * CHUNK + e, 16)]
                i2 = idx2_v[pl.ds(p * CHUNK + e, 16)]
                for g in range(NP):
                    bf = jnp.bfloat16
                    xA1 = plsc.bitcast(plsc.load_gather(tA[g], [i1]), bf)
                    xA2 = plsc.bitcast(plsc.load_gather(tA[g], [i2]), bf)
                    xB1 = plsc.bitcast(plsc.load_gather(tB[g], [i1]), bf)
                    xB2 = plsc.bitcast(plsc.load_gather(tB[g], [i2]), bf)
                    xC1 = plsc.bitcast(plsc.load_gather(tC[g], [i1]), bf)
                    xC2 = plsc.bitcast(plsc.load_gather(tC[g], [i2]), bf)
                    nm = xA1 * xB2 + xA2 * xB1
                    dn = xC1 + xC2
                    n0, n1 = plsc.unpack(nm, format=plsc.PackFormat.INTERLEAVED)
                    d0, d1 = plsc.unpack(dn, format=plsc.PackFormat.INTERLEAVED)
                    r = 2 * g
                    out_v[pl.ds((p * RW + r) * CHUNK + e, 16)] = n0 / d0
                    out_v[pl.ds((p * RW + r + 1) * CHUNK + e, 16)] = n1 / d1

            for cp in out_copies(c, p):
                cp.start()
        return 0

    lax.fori_loop(0, NCHUNKS // 2, pair_body, 0)

    for p in (0, 1):
        for cp in out_copies(NCHUNKS - 2 + p, p):
            cp.wait()


def kernel(m1, m2, polar, indices):
    ind1 = indices[0, :].astype(jnp.int32)
    ind2 = indices[1, :].astype(jnp.int32)
    mesh = plsc.VectorSubcoreMesh(core_axis_name="c", subcore_axis_name="s")
    f = pl.kernel(
        _sc_body,
        out_type=jax.ShapeDtypeStruct((D, E), jnp.float32),
        mesh=mesh,
        compiler_params=pltpu.CompilerParams(needs_layout_passes=False,
                                             use_tc_tiling_on_sc=False),
        scratch_types=(
            [pltpu.VMEM((N,), jnp.float32) for _ in range(3 * NP)]
            + [
                pltpu.VMEM((2 * CHUNK,), jnp.int32),
                pltpu.VMEM((2 * CHUNK,), jnp.int32),
                pltpu.VMEM((2 * RW * CHUNK,), jnp.float32),
                pltpu.SemaphoreType.DMA,
                pltpu.SemaphoreType.DMA,
                pltpu.SemaphoreType.DMA,
                pltpu.SemaphoreType.DMA,
            ]
        ),
    )
    return f(m1, m2, polar, ind1, ind2)
